# baseline (speedup 1.0000x reference)
import functools

import numpy as np
import jax
import jax.numpy as jnp
from jax import lax
from jax.experimental import pallas as pl
from jax.experimental.pallas import tpu as pltpu
from jax.experimental.pallas import tpu_sc as plsc

N_NODES = 100000
DEGREE = 16
D_IN = 128
BATCH = 4096
NSAMP = 10
D_OUT = 256

def _sample_cols(layer):
    perm = jax.random.permutation(
        jax.random.fold_in(jax.random.key(1234), layer), DEGREE
    )[:NSAMP].astype(jnp.int32)
    return jnp.concatenate([perm, jnp.zeros((16 - NSAMP,), jnp.int32)])

_NC = 2
_NS = 16
_NW = _NC * _NS
_SPW = BATCH // _NW


def _sc_gather(ids, adj, feats, cols0, cols1):
    mesh = plsc.VectorSubcoreMesh(core_axis_name="c", subcore_axis_name="s")
    S = _SPW

    def body(ids_hbm, adj_hbm, feats_hbm, c0_hbm, c1_hbm, x0_hbm, x1_hbm, s2_hbm,
             sid_v, adjrows_v, a2big, n1_v, n2_v, c0_v, c1_v,
             x1bufs, fbufs, accbufs,
             sem_g, sem_f0, sem_f1, sem_w, sem_x):
        wid = lax.axis_index("s") * _NC + lax.axis_index("c")
        base = wid * S
        lanes = lax.iota(jnp.int32, 16)
        fsems = (sem_f0, sem_f1)

        pltpu.sync_copy(c0_hbm, c0_v)
        pltpu.sync_copy(c1_hbm, c1_v)
        pltpu.sync_copy(ids_hbm.at[pl.ds(base, S)], sid_v)
        cp_adj = pltpu.async_copy(adj_hbm.at[sid_v], adjrows_v, sem_g)
        cp_x0 = pltpu.async_copy(feats_hbm.at[sid_v], fbufs.at[0], sem_f0)
        cp_adj.wait()

        c0 = c0_v[...]
        c1 = c1_v[...]
        for j in range(NSAMP):
            col = jnp.full((16,), c0[j], jnp.int32)
            for i in range(S // 16):
                rows = lanes + (i * 16)
                n1_v[j, pl.ds(i * 16, 16)] = plsc.load_gather(adjrows_v, [rows, col])

        cp_x0.wait()
        cp_x0w = pltpu.async_copy(fbufs.at[0], x0_hbm.at[pl.ds(base, S)], sem_w)

        H = NSAMP // 2

        def build_n2(j, src):
            for k in range(NSAMP):
                colk = jnp.full((16,), c1[k], jnp.int32)
                for i in range(S // 16):
                    rows = lanes + (i * 16)
                    n2_v[j * NSAMP + k, pl.ds(i * 16, 16)] = plsc.load_gather(
                        src, [rows, colk]
                    )

        for half in range(2):
            ds = [
                pltpu.async_copy(
                    adj_hbm.at[n1_v.at[half * H + jj]], a2big.at[jj], sem_g
                )
                for jj in range(H)
            ]
            for d in ds:
                d.wait()
            for jj in range(H):
                build_n2(half * H + jj, a2big.at[jj])
        cp_x0w.wait()

        def do_j(j, par):
            cp_x1 = pltpu.async_copy(feats_hbm.at[n1_v.at[j]], x1bufs.at[par], sem_x)
            acc = accbufs.at[par]
            cp_k0 = pltpu.async_copy(feats_hbm.at[n2_v.at[j * NSAMP]], acc, sem_g)
            cps = [None, None]
            cps[0] = pltpu.async_copy(
                feats_hbm.at[n2_v.at[j * NSAMP + 1]], fbufs.at[0], sem_f0
            )
            cps[1] = pltpu.async_copy(
                feats_hbm.at[n2_v.at[j * NSAMP + 2]], fbufs.at[1], sem_f1
            )
            cp_x1.wait()
            w_x1 = pltpu.async_copy(x1bufs.at[par], x1_hbm.at[j, pl.ds(base, S)], sem_w)
            cp_k0.wait()
            for k in range(1, NSAMP):
                p = (k - 1) % 2
                cps[p].wait()
                buf = fbufs.at[p]

                @plsc.parallel_loop(0, S, unroll=4)
                def racc(r):
                    for c in range(D_IN // 16):
                        sl = pl.ds(c * 16, 16)
                        plsc.addupdate(acc.at[r, sl], buf[r, sl])

                if k + 2 < NSAMP:
                    cps[p] = pltpu.async_copy(
                        feats_hbm.at[n2_v.at[j * NSAMP + k + 2]], fbufs.at[p], fsems[p]
                    )
            w_acc = pltpu.async_copy(acc, s2_hbm.at[j, pl.ds(base, S)], sem_w)
            return w_x1, w_acc

        @pl.loop(0, NSAMP, step=2)
        def jpair(j):
            w1a, w2a = do_j(j, 0)
            w1b, w2b = do_j(j + 1, 1)
            w1a.wait()
            w2a.wait()
            w1b.wait()
            w2b.wait()

    f = pl.kernel(
        body,
        out_type=(
            jax.ShapeDtypeStruct((BATCH, D_IN), jnp.float32),
            jax.ShapeDtypeStruct((NSAMP, BATCH, D_IN), jnp.float32),
            jax.ShapeDtypeStruct((NSAMP, BATCH, D_IN), jnp.float32),
        ),
        mesh=mesh,
        compiler_params=pltpu.CompilerParams(
            needs_layout_passes=False, use_tc_tiling_on_sc=False
        ),
        scratch_types=[
            pltpu.VMEM((S,), jnp.int32),
            pltpu.VMEM((S, DEGREE), jnp.int32),
            pltpu.VMEM((NSAMP // 2, S, DEGREE), jnp.int32),
            pltpu.VMEM((NSAMP, S), jnp.int32),
            pltpu.VMEM((NSAMP * NSAMP, S), jnp.int32),
            pltpu.VMEM((16,), jnp.int32),
            pltpu.VMEM((16,), jnp.int32),
            pltpu.VMEM((2, S, D_IN), jnp.float32),
            pltpu.VMEM((2, S, D_IN), jnp.float32),
            pltpu.VMEM((2, S, D_IN), jnp.float32),
            pltpu.SemaphoreType.DMA,
            pltpu.SemaphoreType.DMA,
            pltpu.SemaphoreType.DMA,
            pltpu.SemaphoreType.DMA,
            pltpu.SemaphoreType.DMA,
        ],
    )
    return f(ids, adj, feats, cols0, cols1)


def _tc_layer1(x1, s2, Wx1, bx1, Wn1, bn1):
    SB = 512

    def body(x1_ref, s2_ref, wx_ref, bx_ref, wn_ref, bn_ref, agg_ref, sx_ref):
        wx = wx_ref[...]
        bx = bx_ref[...]
        wn = wn_ref[...] * (1.0 / NSAMP)
        bn = bn_ref[...]
        acc = jnp.zeros((SB, D_OUT), jnp.float32)
        xs = jnp.zeros((SB, D_IN), jnp.float32)
        for j in range(NSAMP):
            xj = x1_ref[j]
            aj = s2_ref[j]
            hx = jnp.dot(xj, wx, preferred_element_type=jnp.float32) + bx
            hn = jnp.dot(aj, wn, preferred_element_type=jnp.float32) + bn
            acc += jax.nn.relu(jnp.concatenate([hx, hn], axis=1))
            xs += xj
        agg_ref[...] = acc
        sx_ref[...] = xs

    return pl.pallas_call(
        body,
        grid=(BATCH // SB,),
        in_specs=[
            pl.BlockSpec((NSAMP, SB, D_IN), lambda i: (0, i, 0)),
            pl.BlockSpec((NSAMP, SB, D_IN), lambda i: (0, i, 0)),
            pl.BlockSpec((D_IN, D_IN), lambda i: (0, 0)),
            pl.BlockSpec((1, D_IN), lambda i: (0, 0)),
            pl.BlockSpec((D_IN, D_IN), lambda i: (0, 0)),
            pl.BlockSpec((1, D_IN), lambda i: (0, 0)),
        ],
        out_specs=[
            pl.BlockSpec((SB, D_OUT), lambda i: (i, 0)),
            pl.BlockSpec((SB, D_IN), lambda i: (i, 0)),
        ],
        out_shape=[
            jax.ShapeDtypeStruct((BATCH, D_OUT), jnp.float32),
            jax.ShapeDtypeStruct((BATCH, D_IN), jnp.float32),
        ],
    )(x1, s2, Wx1, bx1, Wn1, bn1)


def _tc_layer2(x0, sx1, agg1s, Wx1, bx1, Wn1, bn1, Wx2, bx2, Wn2, bn2):
    SB = 1024

    def body(x0_ref, sx_ref, ag_ref, wx1_ref, bx1_ref, wn1_ref, bn1_ref,
             wx2_ref, bx2_ref, wn2_ref, bn2_ref, out_ref):
        x0b = x0_ref[...]
        sxb = sx_ref[...]
        agb = ag_ref[...]
        wn1 = wn1_ref[...] * (1.0 / NSAMP)
        wn2 = wn2_ref[...] * (1.0 / NSAMP)
        hx = jnp.dot(x0b, wx1_ref[...], preferred_element_type=jnp.float32) + bx1_ref[...]
        hn = jnp.dot(sxb, wn1, preferred_element_type=jnp.float32) + bn1_ref[...]
        h0 = jax.nn.relu(jnp.concatenate([hx, hn], axis=1))
        ox = jnp.dot(h0, wx2_ref[...], preferred_element_type=jnp.float32) + bx2_ref[...]
        on = jnp.dot(agb, wn2, preferred_element_type=jnp.float32) + bn2_ref[...]
        out_ref[...] = jnp.concatenate([ox, on], axis=1)

    return pl.pallas_call(
        body,
        grid=(BATCH // SB,),
        in_specs=[
            pl.BlockSpec((SB, D_IN), lambda i: (i, 0)),
            pl.BlockSpec((SB, D_IN), lambda i: (i, 0)),
            pl.BlockSpec((SB, D_OUT), lambda i: (i, 0)),
            pl.BlockSpec((D_IN, D_IN), lambda i: (0, 0)),
            pl.BlockSpec((1, D_IN), lambda i: (0, 0)),
            pl.BlockSpec((D_IN, D_IN), lambda i: (0, 0)),
            pl.BlockSpec((1, D_IN), lambda i: (0, 0)),
            pl.BlockSpec((D_OUT, D_IN), lambda i: (0, 0)),
            pl.BlockSpec((1, D_IN), lambda i: (0, 0)),
            pl.BlockSpec((D_OUT, D_IN), lambda i: (0, 0)),
            pl.BlockSpec((1, D_IN), lambda i: (0, 0)),
        ],
        out_specs=pl.BlockSpec((SB, D_OUT), lambda i: (i, 0)),
        out_shape=jax.ShapeDtypeStruct((BATCH, D_OUT), jnp.float32),
    )(x0, sx1, agg1s, Wx1, bx1, Wn1, bn1, Wx2, bx2, Wn2, bn2)


def kernel(ids, adj, feats, W_x1, b_x1, W_n1, b_n1, W_x2, b_x2, W_n2, b_n2):
    adj = jnp.bitwise_and(adj, jnp.int32(0x7FFFFFFF))
    x0, x1, s2 = _sc_gather(ids, adj, feats, _sample_cols(0), _sample_cols(1))
    agg1s, sx1 = _tc_layer1(x1, s2, W_x1, b_x1.reshape(1, -1), W_n1,
                            b_n1.reshape(1, -1))
    return _tc_layer2(x0, sx1, agg1s, W_x1, b_x1.reshape(1, -1), W_n1,
                      b_n1.reshape(1, -1), W_x2, b_x2.reshape(1, -1), W_n2,
                      b_n2.reshape(1, -1))


# speedup vs baseline: 1.1303x; 1.1303x over previous
import functools

import numpy as np
import jax
import jax.numpy as jnp
from jax import lax
from jax.experimental import pallas as pl
from jax.experimental.pallas import tpu as pltpu
from jax.experimental.pallas import tpu_sc as plsc

N_NODES = 100000
DEGREE = 16
D_IN = 128
BATCH = 4096
NSAMP = 10
D_OUT = 256

def _sample_cols(layer):
    perm = jax.random.permutation(
        jax.random.fold_in(jax.random.key(1234), layer), DEGREE
    )[:NSAMP].astype(jnp.int32)
    return jnp.concatenate([perm, jnp.zeros((16 - NSAMP,), jnp.int32)])

_NC = 2
_NS = 16
_NW = _NC * _NS
_SPW = BATCH // _NW


def _sc_gather(ids, adj, feats, cols0, cols1):
    mesh = plsc.VectorSubcoreMesh(core_axis_name="c", subcore_axis_name="s")
    S = _SPW

    def body(ids_hbm, adj_hbm, feats_hbm, c0_hbm, c1_hbm, x0_hbm, x1_hbm, s2_hbm,
             sid_v, adjrows_v, a2big, n1_v, n2_v, c0_v, c1_v,
             x1bufs, fbufs, accbufs,
             sem_g, sem_f0, sem_f1, sem_w, sem_x):
        wid = lax.axis_index("s") * _NC + lax.axis_index("c")
        base = wid * S
        lanes = lax.iota(jnp.int32, 16)
        fsems = (sem_f0, sem_f1)

        pltpu.sync_copy(c0_hbm, c0_v)
        pltpu.sync_copy(c1_hbm, c1_v)
        pltpu.sync_copy(ids_hbm.at[pl.ds(base, S)], sid_v)
        cp_adj = pltpu.async_copy(adj_hbm.at[sid_v], adjrows_v, sem_g)
        cp_x0 = pltpu.async_copy(feats_hbm.at[sid_v], fbufs.at[0], sem_f0)
        cp_adj.wait()

        c0 = c0_v[...]
        c1 = c1_v[...]
        for j in range(NSAMP):
            col = jnp.full((16,), c0[j], jnp.int32)
            for i in range(S // 16):
                rows = lanes + (i * 16)
                n1_v[j, pl.ds(i * 16, 16)] = plsc.load_gather(adjrows_v, [rows, col])

        cp_x0.wait()
        cp_x0w = pltpu.async_copy(fbufs.at[0], x0_hbm.at[pl.ds(base, S)], sem_w)

        H = NSAMP // 2

        def build_n2(j, src):
            for k in range(NSAMP):
                colk = jnp.full((16,), c1[k], jnp.int32)
                for i in range(S // 16):
                    rows = lanes + (i * 16)
                    n2_v[j * NSAMP + k, pl.ds(i * 16, 16)] = plsc.load_gather(
                        src, [rows, colk]
                    )

        for half in range(2):
            ds = [
                pltpu.async_copy(
                    adj_hbm.at[n1_v.at[half * H + jj]], a2big.at[jj], sem_g
                )
                for jj in range(H)
            ]
            for d in ds:
                d.wait()
            for jj in range(H):
                build_n2(half * H + jj, a2big.at[jj])
        cp_x0w.wait()

        def do_j(j, par):
            cp_x1 = pltpu.async_copy(feats_hbm.at[n1_v.at[j]], x1bufs.at[par], sem_x)
            acc = accbufs.at[par]
            cp_k0 = pltpu.async_copy(feats_hbm.at[n2_v.at[j * NSAMP]], acc, sem_g)
            cps = [None, None]
            cps[0] = pltpu.async_copy(
                feats_hbm.at[n2_v.at[j * NSAMP + 1]], fbufs.at[0], sem_f0
            )
            cps[1] = pltpu.async_copy(
                feats_hbm.at[n2_v.at[j * NSAMP + 2]], fbufs.at[1], sem_f1
            )
            cp_x1.wait()
            w_x1 = pltpu.async_copy(x1bufs.at[par], x1_hbm.at[j, pl.ds(base, S)], sem_w)
            cp_k0.wait()
            for k in range(1, NSAMP):
                p = (k - 1) % 2
                cps[p].wait()
                buf = fbufs.at[p]

                @pl.loop(0, S, unroll=2)
                def racc(r):
                    for c in range(D_IN // 16):
                        sl = pl.ds(c * 16, 16)
                        plsc.addupdate(acc.at[r, sl], buf[r, sl])

                if k + 2 < NSAMP:
                    cps[p] = pltpu.async_copy(
                        feats_hbm.at[n2_v.at[j * NSAMP + k + 2]], fbufs.at[p], fsems[p]
                    )
            w_acc = pltpu.async_copy(acc, s2_hbm.at[j, pl.ds(base, S)], sem_w)
            return w_x1, w_acc

        @pl.loop(0, NSAMP, step=2)
        def jpair(j):
            w1a, w2a = do_j(j, 0)
            w1b, w2b = do_j(j + 1, 1)
            w1a.wait()
            w2a.wait()
            w1b.wait()
            w2b.wait()

    f = pl.kernel(
        body,
        out_type=(
            jax.ShapeDtypeStruct((BATCH, D_IN), jnp.float32),
            jax.ShapeDtypeStruct((NSAMP, BATCH, D_IN), jnp.float32),
            jax.ShapeDtypeStruct((NSAMP, BATCH, D_IN), jnp.float32),
        ),
        mesh=mesh,
        compiler_params=pltpu.CompilerParams(
            needs_layout_passes=False, use_tc_tiling_on_sc=False
        ),
        scratch_types=[
            pltpu.VMEM((S,), jnp.int32),
            pltpu.VMEM((S, DEGREE), jnp.int32),
            pltpu.VMEM((NSAMP // 2, S, DEGREE), jnp.int32),
            pltpu.VMEM((NSAMP, S), jnp.int32),
            pltpu.VMEM((NSAMP * NSAMP, S), jnp.int32),
            pltpu.VMEM((16,), jnp.int32),
            pltpu.VMEM((16,), jnp.int32),
            pltpu.VMEM((2, S, D_IN), jnp.float32),
            pltpu.VMEM((2, S, D_IN), jnp.float32),
            pltpu.VMEM((2, S, D_IN), jnp.float32),
            pltpu.SemaphoreType.DMA,
            pltpu.SemaphoreType.DMA,
            pltpu.SemaphoreType.DMA,
            pltpu.SemaphoreType.DMA,
            pltpu.SemaphoreType.DMA,
        ],
    )
    return f(ids, adj, feats, cols0, cols1)


def _tc_layer1(x1, s2, Wx1, bx1, Wn1, bn1):
    SB = 512

    def body(x1_ref, s2_ref, wx_ref, bx_ref, wn_ref, bn_ref, agg_ref, sx_ref):
        wx = wx_ref[...]
        bx = bx_ref[...]
        wn = wn_ref[...] * (1.0 / NSAMP)
        bn = bn_ref[...]
        acc = jnp.zeros((SB, D_OUT), jnp.float32)
        xs = jnp.zeros((SB, D_IN), jnp.float32)
        for j in range(NSAMP):
            xj = x1_ref[j]
            aj = s2_ref[j]
            hx = jnp.dot(xj, wx, preferred_element_type=jnp.float32) + bx
            hn = jnp.dot(aj, wn, preferred_element_type=jnp.float32) + bn
            acc += jax.nn.relu(jnp.concatenate([hx, hn], axis=1))
            xs += xj
        agg_ref[...] = acc
        sx_ref[...] = xs

    return pl.pallas_call(
        body,
        grid=(BATCH // SB,),
        in_specs=[
            pl.BlockSpec((NSAMP, SB, D_IN), lambda i: (0, i, 0)),
            pl.BlockSpec((NSAMP, SB, D_IN), lambda i: (0, i, 0)),
            pl.BlockSpec((D_IN, D_IN), lambda i: (0, 0)),
            pl.BlockSpec((1, D_IN), lambda i: (0, 0)),
            pl.BlockSpec((D_IN, D_IN), lambda i: (0, 0)),
            pl.BlockSpec((1, D_IN), lambda i: (0, 0)),
        ],
        out_specs=[
            pl.BlockSpec((SB, D_OUT), lambda i: (i, 0)),
            pl.BlockSpec((SB, D_IN), lambda i: (i, 0)),
        ],
        out_shape=[
            jax.ShapeDtypeStruct((BATCH, D_OUT), jnp.float32),
            jax.ShapeDtypeStruct((BATCH, D_IN), jnp.float32),
        ],
    )(x1, s2, Wx1, bx1, Wn1, bn1)


def _tc_layer2(x0, sx1, agg1s, Wx1, bx1, Wn1, bn1, Wx2, bx2, Wn2, bn2):
    SB = 1024

    def body(x0_ref, sx_ref, ag_ref, wx1_ref, bx1_ref, wn1_ref, bn1_ref,
             wx2_ref, bx2_ref, wn2_ref, bn2_ref, out_ref):
        x0b = x0_ref[...]
        sxb = sx_ref[...]
        agb = ag_ref[...]
        wn1 = wn1_ref[...] * (1.0 / NSAMP)
        wn2 = wn2_ref[...] * (1.0 / NSAMP)
        hx = jnp.dot(x0b, wx1_ref[...], preferred_element_type=jnp.float32) + bx1_ref[...]
        hn = jnp.dot(sxb, wn1, preferred_element_type=jnp.float32) + bn1_ref[...]
        h0 = jax.nn.relu(jnp.concatenate([hx, hn], axis=1))
        ox = jnp.dot(h0, wx2_ref[...], preferred_element_type=jnp.float32) + bx2_ref[...]
        on = jnp.dot(agb, wn2, preferred_element_type=jnp.float32) + bn2_ref[...]
        out_ref[...] = jnp.concatenate([ox, on], axis=1)

    return pl.pallas_call(
        body,
        grid=(BATCH // SB,),
        in_specs=[
            pl.BlockSpec((SB, D_IN), lambda i: (i, 0)),
            pl.BlockSpec((SB, D_IN), lambda i: (i, 0)),
            pl.BlockSpec((SB, D_OUT), lambda i: (i, 0)),
            pl.BlockSpec((D_IN, D_IN), lambda i: (0, 0)),
            pl.BlockSpec((1, D_IN), lambda i: (0, 0)),
            pl.BlockSpec((D_IN, D_IN), lambda i: (0, 0)),
            pl.BlockSpec((1, D_IN), lambda i: (0, 0)),
            pl.BlockSpec((D_OUT, D_IN), lambda i: (0, 0)),
            pl.BlockSpec((1, D_IN), lambda i: (0, 0)),
            pl.BlockSpec((D_OUT, D_IN), lambda i: (0, 0)),
            pl.BlockSpec((1, D_IN), lambda i: (0, 0)),
        ],
        out_specs=pl.BlockSpec((SB, D_OUT), lambda i: (i, 0)),
        out_shape=jax.ShapeDtypeStruct((BATCH, D_OUT), jnp.float32),
    )(x0, sx1, agg1s, Wx1, bx1, Wn1, bn1, Wx2, bx2, Wn2, bn2)


def kernel(ids, adj, feats, W_x1, b_x1, W_n1, b_n1, W_x2, b_x2, W_n2, b_n2):
    x0, x1, s2 = _sc_gather(ids, adj, feats, _sample_cols(0), _sample_cols(1))
    agg1s, sx1 = _tc_layer1(x1, s2, W_x1, b_x1.reshape(1, -1), W_n1,
                            b_n1.reshape(1, -1))
    return _tc_layer2(x0, sx1, agg1s, W_x1, b_x1.reshape(1, -1), W_n1,
                      b_n1.reshape(1, -1), W_x2, b_x2.reshape(1, -1), W_n2,
                      b_n2.reshape(1, -1))


# speedup vs baseline: 1.1575x; 1.0240x over previous
import functools

import numpy as np
import jax
import jax.numpy as jnp
from jax import lax
from jax.experimental import pallas as pl
from jax.experimental.pallas import tpu as pltpu
from jax.experimental.pallas import tpu_sc as plsc

N_NODES = 100000
DEGREE = 16
D_IN = 128
BATCH = 4096
NSAMP = 10
D_OUT = 256

def _sample_cols(layer):
    perm = jax.random.permutation(
        jax.random.fold_in(jax.random.key(1234), layer), DEGREE
    )[:NSAMP].astype(jnp.int32)
    return jnp.concatenate([perm, jnp.zeros((16 - NSAMP,), jnp.int32)])

_NC = 2
_NS = 16
_NW = _NC * _NS
_SPW = BATCH // _NW


def _sc_gather(ids, adj, feats, cols0, cols1):
    mesh = plsc.VectorSubcoreMesh(core_axis_name="c", subcore_axis_name="s")
    S = _SPW

    def body(ids_hbm, adj_hbm, feats_hbm, c0_hbm, c1_hbm, x0_hbm, x1_hbm, s2_hbm,
             sid_v, adjrows_v, a2big, n1_v, n2_v, c0_v, c1_v,
             x1buf, fbufs, accbufs,
             sem_g, sem_f0, sem_f1, sem_f2, sem_w, sem_x):
        wid = lax.axis_index("s") * _NC + lax.axis_index("c")
        base = wid * S
        lanes = lax.iota(jnp.int32, 16)
        fsems = (sem_f0, sem_f1, sem_f2)

        pltpu.sync_copy(c0_hbm, c0_v)
        pltpu.sync_copy(c1_hbm, c1_v)
        pltpu.sync_copy(ids_hbm.at[pl.ds(base, S)], sid_v)
        cp_adj = pltpu.async_copy(adj_hbm.at[sid_v], adjrows_v, sem_g)
        cp_x0 = pltpu.async_copy(feats_hbm.at[sid_v], fbufs.at[0], sem_f0)
        cp_adj.wait()

        c0 = c0_v[...]
        c1 = c1_v[...]
        for j in range(NSAMP):
            col = jnp.full((16,), c0[j], jnp.int32)
            for i in range(S // 16):
                rows = lanes + (i * 16)
                n1_v[j, pl.ds(i * 16, 16)] = plsc.load_gather(adjrows_v, [rows, col])

        cp_x0.wait()
        cp_x0w = pltpu.async_copy(fbufs.at[0], x0_hbm.at[pl.ds(base, S)], sem_w)

        H = NSAMP // 2

        def build_n2(j, src):
            for k in range(NSAMP):
                colk = jnp.full((16,), c1[k], jnp.int32)
                for i in range(S // 16):
                    rows = lanes + (i * 16)
                    n2_v[j * NSAMP + k, pl.ds(i * 16, 16)] = plsc.load_gather(
                        src, [rows, colk]
                    )

        for half in range(2):
            ds = [
                pltpu.async_copy(
                    adj_hbm.at[n1_v.at[half * H + jj]], a2big.at[jj], sem_g
                )
                for jj in range(H)
            ]
            for d in ds:
                d.wait()
            for jj in range(H):
                build_n2(half * H + jj, a2big.at[jj])
        cp_x0w.wait()

        def do_j(j, acc, cp_x1):
            cp_k0 = pltpu.async_copy(feats_hbm.at[n2_v.at[j * NSAMP]], acc, sem_g)
            cps = [
                pltpu.async_copy(
                    feats_hbm.at[n2_v.at[j * NSAMP + 1 + p]], fbufs.at[p], fsems[p]
                )
                for p in range(3)
            ]
            cp_x1.wait()
            w_x1 = pltpu.async_copy(x1buf, x1_hbm.at[j, pl.ds(base, S)], sem_w)
            cp_k0.wait()
            for k in range(1, NSAMP):
                p = (k - 1) % 3
                cps[p].wait()
                buf = fbufs.at[p]

                @pl.loop(0, S, unroll=2)
                def racc(r):
                    for c in range(D_IN // 16):
                        sl = pl.ds(c * 16, 16)
                        plsc.addupdate(acc.at[r, sl], buf[r, sl])

                if k + 3 < NSAMP:
                    cps[p] = pltpu.async_copy(
                        feats_hbm.at[n2_v.at[j * NSAMP + k + 3]], fbufs.at[p], fsems[p]
                    )
            w_acc = pltpu.async_copy(acc, s2_hbm.at[j, pl.ds(base, S)], sem_w)
            return w_x1, w_acc

        @pl.loop(0, NSAMP, step=2)
        def jpair(j):
            cpx_a = pltpu.async_copy(feats_hbm.at[n1_v.at[j]], x1buf, sem_x)
            w1a, w2a = do_j(j, accbufs.at[0], cpx_a)
            w1a.wait()
            cpx_b = pltpu.async_copy(feats_hbm.at[n1_v.at[j + 1]], x1buf, sem_x)
            w1b, w2b = do_j(j + 1, accbufs.at[1], cpx_b)
            w2a.wait()
            w1b.wait()
            w2b.wait()

    f = pl.kernel(
        body,
        out_type=(
            jax.ShapeDtypeStruct((BATCH, D_IN), jnp.float32),
            jax.ShapeDtypeStruct((NSAMP, BATCH, D_IN), jnp.float32),
            jax.ShapeDtypeStruct((NSAMP, BATCH, D_IN), jnp.float32),
        ),
        mesh=mesh,
        compiler_params=pltpu.CompilerParams(
            needs_layout_passes=False, use_tc_tiling_on_sc=False
        ),
        scratch_types=[
            pltpu.VMEM((S,), jnp.int32),
            pltpu.VMEM((S, DEGREE), jnp.int32),
            pltpu.VMEM((NSAMP // 2, S, DEGREE), jnp.int32),
            pltpu.VMEM((NSAMP, S), jnp.int32),
            pltpu.VMEM((NSAMP * NSAMP, S), jnp.int32),
            pltpu.VMEM((16,), jnp.int32),
            pltpu.VMEM((16,), jnp.int32),
            pltpu.VMEM((S, D_IN), jnp.float32),
            pltpu.VMEM((3, S, D_IN), jnp.float32),
            pltpu.VMEM((2, S, D_IN), jnp.float32),
            pltpu.SemaphoreType.DMA,
            pltpu.SemaphoreType.DMA,
            pltpu.SemaphoreType.DMA,
            pltpu.SemaphoreType.DMA,
            pltpu.SemaphoreType.DMA,
            pltpu.SemaphoreType.DMA,
        ],
    )
    return f(ids, adj, feats, cols0, cols1)


def _tc_layer1(x1, s2, Wx1, bx1, Wn1, bn1):
    SB = 512

    def body(x1_ref, s2_ref, wx_ref, bx_ref, wn_ref, bn_ref, agg_ref, sx_ref):
        wx = wx_ref[...]
        bx = bx_ref[...]
        wn = wn_ref[...] * (1.0 / NSAMP)
        bn = bn_ref[...]
        acc = jnp.zeros((SB, D_OUT), jnp.float32)
        xs = jnp.zeros((SB, D_IN), jnp.float32)
        for j in range(NSAMP):
            xj = x1_ref[j]
            aj = s2_ref[j]
            hx = jnp.dot(xj, wx, preferred_element_type=jnp.float32) + bx
            hn = jnp.dot(aj, wn, preferred_element_type=jnp.float32) + bn
            acc += jax.nn.relu(jnp.concatenate([hx, hn], axis=1))
            xs += xj
        agg_ref[...] = acc
        sx_ref[...] = xs

    return pl.pallas_call(
        body,
        grid=(BATCH // SB,),
        in_specs=[
            pl.BlockSpec((NSAMP, SB, D_IN), lambda i: (0, i, 0)),
            pl.BlockSpec((NSAMP, SB, D_IN), lambda i: (0, i, 0)),
            pl.BlockSpec((D_IN, D_IN), lambda i: (0, 0)),
            pl.BlockSpec((1, D_IN), lambda i: (0, 0)),
            pl.BlockSpec((D_IN, D_IN), lambda i: (0, 0)),
            pl.BlockSpec((1, D_IN), lambda i: (0, 0)),
        ],
        out_specs=[
            pl.BlockSpec((SB, D_OUT), lambda i: (i, 0)),
            pl.BlockSpec((SB, D_IN), lambda i: (i, 0)),
        ],
        out_shape=[
            jax.ShapeDtypeStruct((BATCH, D_OUT), jnp.float32),
            jax.ShapeDtypeStruct((BATCH, D_IN), jnp.float32),
        ],
    )(x1, s2, Wx1, bx1, Wn1, bn1)


def _tc_layer2(x0, sx1, agg1s, Wx1, bx1, Wn1, bn1, Wx2, bx2, Wn2, bn2):
    SB = 1024

    def body(x0_ref, sx_ref, ag_ref, wx1_ref, bx1_ref, wn1_ref, bn1_ref,
             wx2_ref, bx2_ref, wn2_ref, bn2_ref, out_ref):
        x0b = x0_ref[...]
        sxb = sx_ref[...]
        agb = ag_ref[...]
        wn1 = wn1_ref[...] * (1.0 / NSAMP)
        wn2 = wn2_ref[...] * (1.0 / NSAMP)
        hx = jnp.dot(x0b, wx1_ref[...], preferred_element_type=jnp.float32) + bx1_ref[...]
        hn = jnp.dot(sxb, wn1, preferred_element_type=jnp.float32) + bn1_ref[...]
        h0 = jax.nn.relu(jnp.concatenate([hx, hn], axis=1))
        ox = jnp.dot(h0, wx2_ref[...], preferred_element_type=jnp.float32) + bx2_ref[...]
        on = jnp.dot(agb, wn2, preferred_element_type=jnp.float32) + bn2_ref[...]
        out_ref[...] = jnp.concatenate([ox, on], axis=1)

    return pl.pallas_call(
        body,
        grid=(BATCH // SB,),
        in_specs=[
            pl.BlockSpec((SB, D_IN), lambda i: (i, 0)),
            pl.BlockSpec((SB, D_IN), lambda i: (i, 0)),
            pl.BlockSpec((SB, D_OUT), lambda i: (i, 0)),
            pl.BlockSpec((D_IN, D_IN), lambda i: (0, 0)),
            pl.BlockSpec((1, D_IN), lambda i: (0, 0)),
            pl.BlockSpec((D_IN, D_IN), lambda i: (0, 0)),
            pl.BlockSpec((1, D_IN), lambda i: (0, 0)),
            pl.BlockSpec((D_OUT, D_IN), lambda i: (0, 0)),
            pl.BlockSpec((1, D_IN), lambda i: (0, 0)),
            pl.BlockSpec((D_OUT, D_IN), lambda i: (0, 0)),
            pl.BlockSpec((1, D_IN), lambda i: (0, 0)),
        ],
        out_specs=pl.BlockSpec((SB, D_OUT), lambda i: (i, 0)),
        out_shape=jax.ShapeDtypeStruct((BATCH, D_OUT), jnp.float32),
    )(x0, sx1, agg1s, Wx1, bx1, Wn1, bn1, Wx2, bx2, Wn2, bn2)


def kernel(ids, adj, feats, W_x1, b_x1, W_n1, b_n1, W_x2, b_x2, W_n2, b_n2):
    x0, x1, s2 = _sc_gather(ids, adj, feats, _sample_cols(0), _sample_cols(1))
    agg1s, sx1 = _tc_layer1(x1, s2, W_x1, b_x1.reshape(1, -1), W_n1,
                            b_n1.reshape(1, -1))
    return _tc_layer2(x0, sx1, agg1s, W_x1, b_x1.reshape(1, -1), W_n1,
                      b_n1.reshape(1, -1), W_x2, b_x2.reshape(1, -1), W_n2,
                      b_n2.reshape(1, -1))


# speedup vs baseline: 1.1880x; 1.0264x over previous
import functools

import numpy as np
import jax
import jax.numpy as jnp
from jax import lax
from jax.experimental import pallas as pl
from jax.experimental.pallas import tpu as pltpu
from jax.experimental.pallas import tpu_sc as plsc

N_NODES = 100000
DEGREE = 16
D_IN = 128
BATCH = 4096
NSAMP = 10
D_OUT = 256

def _sample_cols(layer):
    perm = jax.random.permutation(
        jax.random.fold_in(jax.random.key(1234), layer), DEGREE
    )[:NSAMP].astype(jnp.int32)
    return jnp.concatenate([perm, jnp.zeros((16 - NSAMP,), jnp.int32)])

_NC = 2
_NS = 16
_NW = _NC * _NS
_SPW = BATCH // _NW


def _sc_gather(ids, adj, feats, cols0, cols1):
    mesh = plsc.VectorSubcoreMesh(core_axis_name="c", subcore_axis_name="s")
    S = _SPW

    def body(ids_hbm, adj_hbm, feats_hbm, c0_hbm, c1_hbm, x0_hbm, x1_hbm, s2_hbm,
             sid_v, adjrows_v, a2big, n1_v, n2_v, c0_v, c1_v,
             x1buf, fbufs, accbufs,
             sem_g, sem_f0, sem_f1, sem_f2, sem_w, sem_x):
        wid = lax.axis_index("s") * _NC + lax.axis_index("c")
        base = wid * S
        lanes = lax.iota(jnp.int32, 16)
        fsems = (sem_f0, sem_f1, sem_f2)

        pltpu.sync_copy(c0_hbm, c0_v)
        pltpu.sync_copy(c1_hbm, c1_v)
        pltpu.sync_copy(ids_hbm.at[pl.ds(base, S)], sid_v)
        cp_adj = pltpu.async_copy(adj_hbm.at[sid_v], adjrows_v, sem_g)
        cp_x0 = pltpu.async_copy(feats_hbm.at[sid_v], fbufs.at[0], sem_f0)
        cp_adj.wait()

        c0 = c0_v[...]
        c1 = c1_v[...]
        for j in range(NSAMP):
            col = jnp.full((16,), c0[j], jnp.int32)
            for i in range(S // 16):
                rows = lanes + (i * 16)
                n1_v[j, pl.ds(i * 16, 16)] = plsc.load_gather(adjrows_v, [rows, col])

        cp_x0.wait()
        cp_x0w = pltpu.async_copy(fbufs.at[0], x0_hbm.at[pl.ds(base, S)], sem_w)

        H = NSAMP // 2

        def build_n2(j, src):
            for k in range(NSAMP):
                colk = jnp.full((16,), c1[k], jnp.int32)
                for i in range(S // 16):
                    rows = lanes + (i * 16)
                    n2_v[j * NSAMP + k, pl.ds(i * 16, 16)] = plsc.load_gather(
                        src, [rows, colk]
                    )

        for half in range(2):
            ds = [
                pltpu.async_copy(
                    adj_hbm.at[n1_v.at[half * H + jj]], a2big.at[jj], sem_g
                )
                for jj in range(H)
            ]
            for d in ds:
                d.wait()
            for jj in range(H):
                build_n2(half * H + jj, a2big.at[jj])
        cp_x0w.wait()

        def do_j(j, acc, cp_x1):
            cp_k0 = pltpu.async_copy(feats_hbm.at[n2_v.at[j * NSAMP]], acc, sem_g)
            cps = [
                pltpu.async_copy(
                    feats_hbm.at[n2_v.at[j * NSAMP + 1 + p]], fbufs.at[p], fsems[p]
                )
                for p in range(3)
            ]
            cp_x1.wait()
            w_x1 = pltpu.async_copy(x1buf, x1_hbm.at[j, pl.ds(base, S)], sem_w)
            cp_k0.wait()
            for k in range(1, NSAMP):
                p = (k - 1) % 3
                cps[p].wait()
                buf = fbufs.at[p]

                @pl.loop(0, S, unroll=2)
                def racc(r):
                    for c in range(D_IN // 16):
                        sl = pl.ds(c * 16, 16)
                        plsc.addupdate(acc.at[r, sl], buf[r, sl])

                if k + 3 < NSAMP:
                    cps[p] = pltpu.async_copy(
                        feats_hbm.at[n2_v.at[j * NSAMP + k + 3]], fbufs.at[p], fsems[p]
                    )
            w_acc = pltpu.async_copy(acc, s2_hbm.at[j, pl.ds(base, S)], sem_w)
            return w_x1, w_acc

        @pl.loop(0, NSAMP, step=2)
        def jpair(j):
            cpx_a = pltpu.async_copy(feats_hbm.at[n1_v.at[j]], x1buf, sem_x)
            w1a, w2a = do_j(j, accbufs.at[0], cpx_a)
            w1a.wait()
            cpx_b = pltpu.async_copy(feats_hbm.at[n1_v.at[j + 1]], x1buf, sem_x)
            w1b, w2b = do_j(j + 1, accbufs.at[1], cpx_b)
            w2a.wait()
            w1b.wait()
            w2b.wait()

    f = pl.kernel(
        body,
        out_type=(
            jax.ShapeDtypeStruct((BATCH, D_IN), jnp.float32),
            jax.ShapeDtypeStruct((NSAMP, BATCH, D_IN), jnp.float32),
            jax.ShapeDtypeStruct((NSAMP, BATCH, D_IN), jnp.float32),
        ),
        mesh=mesh,
        compiler_params=pltpu.CompilerParams(
            needs_layout_passes=False, use_tc_tiling_on_sc=False
        ),
        scratch_types=[
            pltpu.VMEM((S,), jnp.int32),
            pltpu.VMEM((S, DEGREE), jnp.int32),
            pltpu.VMEM((NSAMP // 2, S, DEGREE), jnp.int32),
            pltpu.VMEM((NSAMP, S), jnp.int32),
            pltpu.VMEM((NSAMP * NSAMP, S), jnp.int32),
            pltpu.VMEM((16,), jnp.int32),
            pltpu.VMEM((16,), jnp.int32),
            pltpu.VMEM((S, D_IN), jnp.float32),
            pltpu.VMEM((3, S, D_IN), jnp.float32),
            pltpu.VMEM((2, S, D_IN), jnp.float32),
            pltpu.SemaphoreType.DMA,
            pltpu.SemaphoreType.DMA,
            pltpu.SemaphoreType.DMA,
            pltpu.SemaphoreType.DMA,
            pltpu.SemaphoreType.DMA,
            pltpu.SemaphoreType.DMA,
        ],
    )
    return f(ids, adj, feats, cols0, cols1)


def _tc_mlp(x0, x1, s2, Wx1, bx1, Wn1, bn1, Wx2, bx2, Wn2, bn2):
    SB = 512

    def body(x0_ref, x1_ref, s2_ref, wx1_ref, bx1_ref, wn1_ref, bn1_ref,
             wx2_ref, bx2_ref, wn2_ref, bn2_ref, out_ref):
        wx1 = wx1_ref[...]
        bx1 = bx1_ref[...]
        wn1 = wn1_ref[...] * (1.0 / NSAMP)
        bn1 = bn1_ref[...]
        acc = jnp.zeros((SB, D_OUT), jnp.float32)
        xs = jnp.zeros((SB, D_IN), jnp.float32)
        for j in range(NSAMP):
            xj = x1_ref[j]
            hx = jnp.dot(xj, wx1, preferred_element_type=jnp.float32) + bx1
            hn = jnp.dot(s2_ref[j], wn1, preferred_element_type=jnp.float32) + bn1
            acc += jax.nn.relu(jnp.concatenate([hx, hn], axis=1))
            xs += xj
        hx0 = jnp.dot(x0_ref[...], wx1, preferred_element_type=jnp.float32) + bx1
        hn0 = jnp.dot(xs * (1.0 / NSAMP), wn1_ref[...],
                      preferred_element_type=jnp.float32) + bn1
        h0 = jax.nn.relu(jnp.concatenate([hx0, hn0], axis=1))
        ox = jnp.dot(h0, wx2_ref[...], preferred_element_type=jnp.float32) + bx2_ref[...]
        on = jnp.dot(acc * (1.0 / NSAMP), wn2_ref[...],
                     preferred_element_type=jnp.float32) + bn2_ref[...]
        out_ref[...] = jnp.concatenate([ox, on], axis=1)

    return pl.pallas_call(
        body,
        grid=(BATCH // SB,),
        in_specs=[
            pl.BlockSpec((SB, D_IN), lambda i: (i, 0)),
            pl.BlockSpec((NSAMP, SB, D_IN), lambda i: (0, i, 0)),
            pl.BlockSpec((NSAMP, SB, D_IN), lambda i: (0, i, 0)),
            pl.BlockSpec((D_IN, D_IN), lambda i: (0, 0)),
            pl.BlockSpec((1, D_IN), lambda i: (0, 0)),
            pl.BlockSpec((D_IN, D_IN), lambda i: (0, 0)),
            pl.BlockSpec((1, D_IN), lambda i: (0, 0)),
            pl.BlockSpec((D_OUT, D_IN), lambda i: (0, 0)),
            pl.BlockSpec((1, D_IN), lambda i: (0, 0)),
            pl.BlockSpec((D_OUT, D_IN), lambda i: (0, 0)),
            pl.BlockSpec((1, D_IN), lambda i: (0, 0)),
        ],
        out_specs=pl.BlockSpec((SB, D_OUT), lambda i: (i, 0)),
        out_shape=jax.ShapeDtypeStruct((BATCH, D_OUT), jnp.float32),
    )(x0, x1, s2, Wx1, bx1, Wn1, bn1, Wx2, bx2, Wn2, bn2)


def kernel(ids, adj, feats, W_x1, b_x1, W_n1, b_n1, W_x2, b_x2, W_n2, b_n2):
    x0, x1, s2 = _sc_gather(ids, adj, feats, _sample_cols(0), _sample_cols(1))
    return _tc_mlp(x0, x1, s2, W_x1, b_x1.reshape(1, -1), W_n1,
                   b_n1.reshape(1, -1), W_x2, b_x2.reshape(1, -1), W_n2,
                   b_n2.reshape(1, -1))
